# Initial kernel scaffold; baseline (speedup 1.0000x reference)
import jax
import jax.numpy as jnp
from jax.experimental import pallas as pl


def kernel(frame_embeddings, ln_w, ln_b, in_w, conv_w, conv_b, xproj_w, dt_w, dt_b, A_log, D_skip, out_w, oproj_w, oproj_b, fln_w, fln_b):
    raise NotImplementedError("write your pallas kernel here")


# speedup vs baseline: 14.9183x; 14.9183x over previous
import jax
import jax.numpy as jnp
from jax.experimental import pallas as pl
from jax.experimental.pallas import tpu as pltpu

L, D, DI, N, K, DTR = 4, 256, 512, 16, 4, 16
B, T = 4, 1024
TC = 128
NCH = T // TC


def _layer_norm(x, w, b):
    m = jnp.mean(x, axis=-1, keepdims=True)
    xc = x - m
    v = jnp.mean(xc * xc, axis=-1, keepdims=True)
    return xc * jax.lax.rsqrt(v + 1e-5) * w + b


def _softplus(z):
    return jnp.maximum(z, 0.0) + jnp.log1p(jnp.exp(-jnp.abs(z)))


def _body(x_ref, lnw_ref, lnb_ref, inw_ref, cw_ref, cb_ref, xw_ref, dtw_ref,
          dtb_ref, alog_ref, dskip_ref, outw_ref, opw_ref, opb_ref, flnw_ref,
          flnb_ref, o_ref,
          u_s, res_s, dt_s, dtu_s, bc_s, upad_s, da_s, dbu_s, hs_s, y_s):
    x = x_ref[0]
    upad_s[0:8] = jnp.zeros((8, DI), jnp.float32)

    for l in range(L):
        xn = _layer_norm(x, lnw_ref[l], lnb_ref[l])
        ur = jnp.dot(xn, inw_ref[l], preferred_element_type=jnp.float32)
        upad_s[8:] = ur[:, :DI]
        res_s[...] = ur[:, DI:]

        cw = cw_ref[l]
        z = (cw[3:4] * upad_s[8:8 + T]
             + cw[2:3] * upad_s[7:7 + T]
             + cw[1:2] * upad_s[6:6 + T]
             + cw[0:1] * upad_s[5:5 + T]) + cb_ref[l]
        u = z * jax.nn.sigmoid(z)
        u_s[...] = u

        proj = jnp.dot(u, xw_ref[l], preferred_element_type=jnp.float32)
        bc_s[...] = proj
        zdt = jnp.dot(proj[:, :DTR], dtw_ref[l],
                      preferred_element_type=jnp.float32) + dtb_ref[l]
        dt = _softplus(zdt)
        dt_s[...] = dt
        dtu_s[...] = dt * u
        At = -jnp.exp(alog_ref[l])

        h = jnp.zeros((N, DI), jnp.float32)
        for c in range(NCH):
            sl = slice(c * TC, (c + 1) * TC)
            dtc = dt_s[sl]
            duc = dtu_s[sl]
            bcc = bc_s[sl]
            for n in range(N):
                da_s[:, n, :] = jnp.exp(dtc * At[n:n + 1])
                dbu_s[:, n, :] = duc * bcc[:, DTR + n:DTR + n + 1]

            def scan_step(t, hh):
                hn = da_s[t] * hh + dbu_s[t]
                hs_s[t] = hn
                return hn

            h = jax.lax.fori_loop(0, TC, scan_step, h)

            yc = jnp.zeros((TC, DI), jnp.float32)
            for n in range(N):
                yc = yc + hs_s[:, n, :] * bcc[:, DTR + N + n:DTR + N + n + 1]
            y_s[sl] = yc

        y = y_s[...] + u_s[...] * dskip_ref[l]
        resv = res_s[...]
        y = y * (resv * jax.nn.sigmoid(resv))
        x = x + jnp.dot(y, outw_ref[l], preferred_element_type=jnp.float32)

    z = jnp.dot(x, opw_ref[...], preferred_element_type=jnp.float32) + opb_ref[...]
    zn = _layer_norm(z, flnw_ref[...], flnb_ref[...])
    o_ref[0] = jnp.mean(zn, axis=0, keepdims=True)


def kernel(frame_embeddings, ln_w, ln_b, in_w, conv_w, conv_b, xproj_w,
           dt_w, dt_b, A_log, D_skip, out_w, oproj_w, oproj_b, fln_w, fln_b):
    cwT = conv_w.transpose(0, 2, 1)
    alogT = A_log.transpose(0, 2, 1)

    whole = lambda arr: pl.BlockSpec(arr.shape, lambda b: (0,) * arr.ndim)
    out = pl.pallas_call(
        _body,
        grid=(B,),
        in_specs=[
            pl.BlockSpec((1, T, D), lambda b: (b, 0, 0)),
            whole(ln_w), whole(ln_b), whole(in_w), whole(cwT), whole(conv_b),
            whole(xproj_w), whole(dt_w), whole(dt_b), whole(alogT),
            whole(D_skip), whole(out_w), whole(oproj_w), whole(oproj_b),
            whole(fln_w), whole(fln_b),
        ],
        out_specs=pl.BlockSpec((1, 1, D), lambda b: (b, 0, 0)),
        out_shape=jax.ShapeDtypeStruct((B, 1, D), jnp.float32),
        scratch_shapes=[
            pltpu.VMEM((T, DI), jnp.float32),
            pltpu.VMEM((T, DI), jnp.float32),
            pltpu.VMEM((T, DI), jnp.float32),
            pltpu.VMEM((T, DI), jnp.float32),
            pltpu.VMEM((T, DTR + 2 * N), jnp.float32),
            pltpu.VMEM((T + 8, DI), jnp.float32),
            pltpu.VMEM((TC, N, DI), jnp.float32),
            pltpu.VMEM((TC, N, DI), jnp.float32),
            pltpu.VMEM((TC, N, DI), jnp.float32),
            pltpu.VMEM((T, DI), jnp.float32),
        ],
        compiler_params=pltpu.CompilerParams(
            dimension_semantics=("parallel",),
            vmem_limit_bytes=56 * 1024 * 1024,
        ),
        name="mamba_encoder_fused",
    )(frame_embeddings, ln_w, ln_b, in_w, cwT, conv_b, xproj_w, dt_w, dt_b,
      alogT, D_skip, out_w, oproj_w, oproj_b, fln_w, fln_b)
    return out.reshape(B, D)


# speedup vs baseline: 15.1404x; 1.0149x over previous
import jax
import jax.numpy as jnp
from jax.experimental import pallas as pl
from jax.experimental.pallas import tpu as pltpu

L, D, DI, N, K, DTR = 4, 256, 512, 16, 4, 16
B, T = 4, 1024
TC = 128
NCH = T // TC


def _layer_norm(x, w, b):
    m = jnp.mean(x, axis=-1, keepdims=True)
    xc = x - m
    v = jnp.mean(xc * xc, axis=-1, keepdims=True)
    return xc * jax.lax.rsqrt(v + 1e-5) * w + b


def _softplus(z):
    return jnp.maximum(z, 0.0) + jnp.log1p(jnp.exp(-jnp.abs(z)))


def _body(x_ref, lnw_ref, lnb_ref, inw_ref, cw_ref, cb_ref, xw_ref, dtw_ref,
          dtb_ref, alog_ref, dskip_ref, outw_ref, opw_ref, opb_ref, flnw_ref,
          flnb_ref, erep_ref, o_ref,
          u_s, res_s, dt_s, dtu_s, bc_s, upad_s, da_s, dbu_s, hs_s, y_s):
    x = x_ref[0]
    upad_s[0:8] = jnp.zeros((8, DI), jnp.float32)

    for l in range(L):
        xn = _layer_norm(x, lnw_ref[l], lnb_ref[l])
        ur = jnp.dot(xn, inw_ref[l], preferred_element_type=jnp.float32)
        upad_s[8:] = ur[:, :DI]
        res_s[...] = ur[:, DI:]

        cw = cw_ref[l]
        z = (cw[3:4] * upad_s[8:8 + T]
             + cw[2:3] * upad_s[7:7 + T]
             + cw[1:2] * upad_s[6:6 + T]
             + cw[0:1] * upad_s[5:5 + T]) + cb_ref[l]
        u = z * jax.nn.sigmoid(z)
        u_s[...] = u

        proj = jnp.dot(u, xw_ref[l], preferred_element_type=jnp.float32)
        bc_s[...] = proj
        zdt = jnp.dot(proj[:, :DTR], dtw_ref[l],
                      preferred_element_type=jnp.float32) + dtb_ref[l]
        dt = _softplus(zdt)
        dt_s[...] = dt
        dtu_s[...] = dt * u
        At = -jnp.exp(alog_ref[l])

        erep = erep_ref[...]
        h = jnp.zeros((N, DI), jnp.float32)
        for c in range(NCH):
            sl = slice(c * TC, (c + 1) * TC)
            dtc = dt_s[sl]
            duc = dtu_s[sl]
            bcc = bc_s[sl]
            da_s[...] = jnp.exp(dtc[:, None, :] * At[None, :, :])
            b3 = jnp.einsum('tn,nmd->tmd', bcc[:, DTR:DTR + N], erep,
                            preferred_element_type=jnp.float32)
            dbu_s[...] = duc[:, None, :] * b3

            def scan_step(i, hh):
                for k in range(4):
                    t = i * 4 + k
                    hh = da_s[t] * hh + dbu_s[t]
                    hs_s[t] = hh
                return hh

            h = jax.lax.fori_loop(0, TC // 4, scan_step, h)

            c3 = jnp.einsum('tn,nmd->tmd', bcc[:, DTR + N:], erep,
                            preferred_element_type=jnp.float32)
            y_s[sl] = jnp.sum(hs_s[...] * c3, axis=1)

        y = y_s[...] + u_s[...] * dskip_ref[l]
        resv = res_s[...]
        y = y * (resv * jax.nn.sigmoid(resv))
        x = x + jnp.dot(y, outw_ref[l], preferred_element_type=jnp.float32)

    z = jnp.dot(x, opw_ref[...], preferred_element_type=jnp.float32) + opb_ref[...]
    zn = _layer_norm(z, flnw_ref[...], flnb_ref[...])
    o_ref[0] = jnp.mean(zn, axis=0, keepdims=True)


def kernel(frame_embeddings, ln_w, ln_b, in_w, conv_w, conv_b, xproj_w,
           dt_w, dt_b, A_log, D_skip, out_w, oproj_w, oproj_b, fln_w, fln_b):
    cwT = conv_w.transpose(0, 2, 1)
    alogT = A_log.transpose(0, 2, 1)
    erep = jnp.broadcast_to(jnp.eye(N, dtype=jnp.float32)[:, :, None],
                            (N, N, DI))

    whole = lambda arr: pl.BlockSpec(arr.shape, lambda b: (0,) * arr.ndim)
    out = pl.pallas_call(
        _body,
        grid=(B,),
        in_specs=[
            pl.BlockSpec((1, T, D), lambda b: (b, 0, 0)),
            whole(ln_w), whole(ln_b), whole(in_w), whole(cwT), whole(conv_b),
            whole(xproj_w), whole(dt_w), whole(dt_b), whole(alogT),
            whole(D_skip), whole(out_w), whole(oproj_w), whole(oproj_b),
            whole(fln_w), whole(fln_b), whole(erep),
        ],
        out_specs=pl.BlockSpec((1, 1, D), lambda b: (b, 0, 0)),
        out_shape=jax.ShapeDtypeStruct((B, 1, D), jnp.float32),
        scratch_shapes=[
            pltpu.VMEM((T, DI), jnp.float32),
            pltpu.VMEM((T, DI), jnp.float32),
            pltpu.VMEM((T, DI), jnp.float32),
            pltpu.VMEM((T, DI), jnp.float32),
            pltpu.VMEM((T, DTR + 2 * N), jnp.float32),
            pltpu.VMEM((T + 8, DI), jnp.float32),
            pltpu.VMEM((TC, N, DI), jnp.float32),
            pltpu.VMEM((TC, N, DI), jnp.float32),
            pltpu.VMEM((TC, N, DI), jnp.float32),
            pltpu.VMEM((T, DI), jnp.float32),
        ],
        compiler_params=pltpu.CompilerParams(
            dimension_semantics=("parallel",),
            vmem_limit_bytes=56 * 1024 * 1024,
        ),
        name="mamba_encoder_fused",
    )(frame_embeddings, ln_w, ln_b, in_w, cwT, conv_b, xproj_w, dt_w, dt_b,
      alogT, D_skip, out_w, oproj_w, oproj_b, fln_w, fln_b, erep)
    return out.reshape(B, D)


# speedup vs baseline: 15.2354x; 1.0063x over previous
import jax
import jax.numpy as jnp
from jax.experimental import pallas as pl
from jax.experimental.pallas import tpu as pltpu

L, D, DI, N, K, DTR = 4, 256, 512, 16, 4, 16
B, T = 4, 1024
TC = 128
NCH = T // TC


def _layer_norm(x, w, b):
    m = jnp.mean(x, axis=-1, keepdims=True)
    xc = x - m
    v = jnp.mean(xc * xc, axis=-1, keepdims=True)
    return xc * jax.lax.rsqrt(v + 1e-5) * w + b


def _softplus(z):
    return jnp.maximum(z, 0.0) + jnp.log1p(jnp.exp(-jnp.abs(z)))


def _body(x_ref, lnw_ref, lnb_ref, inw_ref, cw_ref, cb_ref, xw_ref, dtw_ref,
          dtb_ref, alog_ref, dskip_ref, outw_ref, opw_ref, opb_ref, flnw_ref,
          flnb_ref, erep_ref, o_ref,
          u_s, res_s, dt_s, dtu_s, bc_s, upad_s, da_s, dbu_s, hs_s, y_s):
    x = x_ref[0]
    upad_s[0:8] = jnp.zeros((8, DI), jnp.float32)

    for l in range(L):
        xn = _layer_norm(x, lnw_ref[l], lnb_ref[l])
        ur = jnp.dot(xn, inw_ref[l], preferred_element_type=jnp.float32)
        upad_s[8:] = ur[:, :DI]
        res_s[...] = ur[:, DI:]

        cw = cw_ref[l]
        z = (cw[3:4] * upad_s[8:8 + T]
             + cw[2:3] * upad_s[7:7 + T]
             + cw[1:2] * upad_s[6:6 + T]
             + cw[0:1] * upad_s[5:5 + T]) + cb_ref[l]
        u = z * jax.nn.sigmoid(z)
        u_s[...] = u

        proj = jnp.dot(u, xw_ref[l], preferred_element_type=jnp.float32)
        bc_s[...] = proj
        zdt = jnp.dot(proj[:, :DTR], dtw_ref[l],
                      preferred_element_type=jnp.float32) + dtb_ref[l]
        dt = _softplus(zdt)
        dt_s[...] = dt
        dtu_s[...] = dt * u
        At = -jnp.exp(alog_ref[l])

        erep = erep_ref[...]
        h = jnp.zeros((N, DI), jnp.float32)
        for c in range(NCH):
            sl = slice(c * TC, (c + 1) * TC)
            dtc = dt_s[sl]
            duc = dtu_s[sl]
            bcc = bc_s[sl]
            da_s[...] = jnp.exp(dtc[:, None, :] * At[None, :, :])
            b3 = jnp.einsum('tn,nmd->tmd', bcc[:, DTR:DTR + N], erep,
                            preferred_element_type=jnp.float32)
            dbu_s[...] = duc[:, None, :] * b3

            def scan_step(i, hh):
                for k in range(4):
                    t = i * 4 + k
                    hh = da_s[t] * hh + dbu_s[t]
                    hs_s[t] = hh
                return hh

            h = jax.lax.fori_loop(0, TC // 4, scan_step, h)

            c3 = jnp.einsum('tn,nmd->tmd', bcc[:, DTR + N:], erep,
                            preferred_element_type=jnp.float32)
            y_s[sl] = jnp.sum(hs_s[...] * c3, axis=1)

        y = y_s[...] + u_s[...] * dskip_ref[l]
        resv = res_s[...]
        y = y * (resv * jax.nn.sigmoid(resv))
        x = x + jnp.dot(y, outw_ref[l], preferred_element_type=jnp.float32)

    z = jnp.dot(x, opw_ref[...], preferred_element_type=jnp.float32) + opb_ref[...]
    zn = _layer_norm(z, flnw_ref[...], flnb_ref[...])
    o_ref[0] = jnp.mean(zn, axis=0, keepdims=True)


def kernel(frame_embeddings, ln_w, ln_b, in_w, conv_w, conv_b, xproj_w,
           dt_w, dt_b, A_log, D_skip, out_w, oproj_w, oproj_b, fln_w, fln_b):
    cwT = conv_w.transpose(0, 2, 1)
    alogT = A_log.transpose(0, 2, 1)
    erep = jnp.broadcast_to(jnp.eye(N, dtype=jnp.float32)[:, :, None],
                            (N, N, DI))

    whole = lambda arr: pl.BlockSpec(arr.shape, lambda b: (0,) * arr.ndim)
    out = pl.pallas_call(
        _body,
        grid=(B,),
        in_specs=[
            pl.BlockSpec((1, T, D), lambda b: (b, 0, 0)),
            whole(ln_w), whole(ln_b), whole(in_w), whole(cwT), whole(conv_b),
            whole(xproj_w), whole(dt_w), whole(dt_b), whole(alogT),
            whole(D_skip), whole(out_w), whole(oproj_w), whole(oproj_b),
            whole(fln_w), whole(fln_b), whole(erep),
        ],
        out_specs=pl.BlockSpec((1, 1, D), lambda b: (b, 0, 0)),
        out_shape=jax.ShapeDtypeStruct((B, 1, D), jnp.float32),
        scratch_shapes=[
            pltpu.VMEM((T, DI), jnp.float32),
            pltpu.VMEM((T, DI), jnp.float32),
            pltpu.VMEM((T, DI), jnp.float32),
            pltpu.VMEM((T, DI), jnp.float32),
            pltpu.VMEM((T, DTR + 2 * N), jnp.float32),
            pltpu.VMEM((T + 8, DI), jnp.float32),
            pltpu.VMEM((TC, N, DI), jnp.float32),
            pltpu.VMEM((TC, N, DI), jnp.float32),
            pltpu.VMEM((TC, N, DI), jnp.float32),
            pltpu.VMEM((T, DI), jnp.float32),
        ],
        compiler_params=pltpu.CompilerParams(
            dimension_semantics=("arbitrary",),
            vmem_limit_bytes=56 * 1024 * 1024,
        ),
        name="mamba_encoder_fused",
    )(frame_embeddings, ln_w, ln_b, in_w, cwT, conv_b, xproj_w, dt_w, dt_b,
      alogT, D_skip, out_w, oproj_w, oproj_b, fln_w, fln_b, erep)
    return out.reshape(B, D)


# speedup vs baseline: 16.4620x; 1.0805x over previous
import jax
import jax.numpy as jnp
from jax.experimental import pallas as pl
from jax.experimental.pallas import tpu as pltpu

L, D, DI, N, K, DTR = 4, 256, 512, 16, 4, 16
B, T = 4, 1024
BG = 2
TT = BG * T
TC = 64
NCH = T // TC


def _layer_norm(x, w, b):
    m = jnp.mean(x, axis=-1, keepdims=True)
    xc = x - m
    v = jnp.mean(xc * xc, axis=-1, keepdims=True)
    return xc * jax.lax.rsqrt(v + 1e-5) * w + b


def _softplus(z):
    return jnp.maximum(z, 0.0) + jnp.log1p(jnp.exp(-jnp.abs(z)))


def _body(x_ref, lnw_ref, lnb_ref, inw_ref, cw_ref, cb_ref, xw_ref, dtw_ref,
          dtb_ref, alog_ref, dskip_ref, outw_ref, opw_ref, opb_ref, flnw_ref,
          flnb_ref, erep_ref, o_ref,
          u_s, res_s, dt_s, bc_s, upad_s, da_s, dbu_s, hs_s, y_s):
    x = x_ref[...].reshape(TT, D)
    upad_s[:, 0:8, :] = jnp.zeros((BG, 8, DI), jnp.float32)
    erep = erep_ref[...]

    for l in range(L):
        xn = _layer_norm(x, lnw_ref[l], lnb_ref[l])
        ur = jnp.dot(xn, inw_ref[l], preferred_element_type=jnp.float32)
        upad_s[:, 8:, :] = ur[:, :DI].reshape(BG, T, DI)
        res_s[...] = ur[:, DI:]

        cw = cw_ref[l]
        z3 = (cw[3] * upad_s[:, 8:8 + T, :]
              + cw[2] * upad_s[:, 7:7 + T, :]
              + cw[1] * upad_s[:, 6:6 + T, :]
              + cw[0] * upad_s[:, 5:5 + T, :]) + cb_ref[l]
        z = z3.reshape(TT, DI)
        u = z * jax.nn.sigmoid(z)
        u_s[...] = u.reshape(BG, T, DI)

        proj = jnp.dot(u, xw_ref[l], preferred_element_type=jnp.float32)
        bc_s[...] = proj.reshape(BG, T, DTR + 2 * N)
        zdt = jnp.dot(proj[:, :DTR], dtw_ref[l],
                      preferred_element_type=jnp.float32) + dtb_ref[l]
        dt_s[...] = _softplus(zdt).reshape(BG, T, DI)
        At = -jnp.exp(alog_ref[l])

        h = jnp.zeros((BG * N, DI), jnp.float32)
        for c in range(NCH):
            sl = slice(c * TC, (c + 1) * TC)
            dtc = dt_s[:, sl, :]
            duc = dtc * u_s[:, sl, :]
            bcc = bc_s[:, sl, :]
            da4 = jnp.exp(dtc[:, :, None, :] * At[None, None, :, :])
            da_s[:, 0:N, :] = da4[0]
            da_s[:, N:2 * N, :] = da4[1]
            b4 = jnp.einsum('btn,nmd->btmd', bcc[:, :, DTR:DTR + N], erep,
                            preferred_element_type=jnp.float32)
            dbu4 = duc[:, :, None, :] * b4
            dbu_s[:, 0:N, :] = dbu4[0]
            dbu_s[:, N:2 * N, :] = dbu4[1]

            def scan_step(i, hh):
                for k in range(4):
                    t = i * 4 + k
                    hh = da_s[t] * hh + dbu_s[t]
                    hs_s[t] = hh
                return hh

            h = jax.lax.fori_loop(0, TC // 4, scan_step, h)

            c4 = jnp.einsum('btn,nmd->btmd', bcc[:, :, DTR + N:], erep,
                            preferred_element_type=jnp.float32)
            hsv = hs_s[...]
            y_s[0, sl, :] = jnp.sum(hsv[:, 0:N, :] * c4[0], axis=1)
            y_s[1, sl, :] = jnp.sum(hsv[:, N:2 * N, :] * c4[1], axis=1)

        y = y_s[...].reshape(TT, DI) + u_s[...].reshape(TT, DI) * dskip_ref[l]
        resv = res_s[...]
        y = y * (resv * jax.nn.sigmoid(resv))
        x = x + jnp.dot(y, outw_ref[l], preferred_element_type=jnp.float32)

    z = jnp.dot(x, opw_ref[...], preferred_element_type=jnp.float32) + opb_ref[...]
    zn = _layer_norm(z, flnw_ref[...], flnb_ref[...])
    o_ref[...] = jnp.mean(zn.reshape(BG, T, D), axis=1, keepdims=True)


def kernel(frame_embeddings, ln_w, ln_b, in_w, conv_w, conv_b, xproj_w,
           dt_w, dt_b, A_log, D_skip, out_w, oproj_w, oproj_b, fln_w, fln_b):
    cwT = conv_w.transpose(0, 2, 1)
    alogT = A_log.transpose(0, 2, 1)
    erep = jnp.broadcast_to(jnp.eye(N, dtype=jnp.float32)[:, :, None],
                            (N, N, DI))

    whole = lambda arr: pl.BlockSpec(arr.shape, lambda b: (0,) * arr.ndim)
    out = pl.pallas_call(
        _body,
        grid=(B // BG,),
        in_specs=[
            pl.BlockSpec((BG, T, D), lambda b: (b, 0, 0)),
            whole(ln_w), whole(ln_b), whole(in_w), whole(cwT), whole(conv_b),
            whole(xproj_w), whole(dt_w), whole(dt_b), whole(alogT),
            whole(D_skip), whole(out_w), whole(oproj_w), whole(oproj_b),
            whole(fln_w), whole(fln_b), whole(erep),
        ],
        out_specs=pl.BlockSpec((BG, 1, D), lambda b: (b, 0, 0)),
        out_shape=jax.ShapeDtypeStruct((B, 1, D), jnp.float32),
        scratch_shapes=[
            pltpu.VMEM((BG, T, DI), jnp.float32),
            pltpu.VMEM((TT, DI), jnp.float32),
            pltpu.VMEM((BG, T, DI), jnp.float32),
            pltpu.VMEM((BG, T, DTR + 2 * N), jnp.float32),
            pltpu.VMEM((BG, T + 8, DI), jnp.float32),
            pltpu.VMEM((TC, BG * N, DI), jnp.float32),
            pltpu.VMEM((TC, BG * N, DI), jnp.float32),
            pltpu.VMEM((TC, BG * N, DI), jnp.float32),
            pltpu.VMEM((BG, T, DI), jnp.float32),
        ],
        compiler_params=pltpu.CompilerParams(
            dimension_semantics=("arbitrary",),
            vmem_limit_bytes=56 * 1024 * 1024,
        ),
        name="mamba_encoder_fused",
    )(frame_embeddings, ln_w, ln_b, in_w, cwT, conv_b, xproj_w, dt_w, dt_b,
      alogT, D_skip, out_w, oproj_w, oproj_b, fln_w, fln_b, erep)
    return out.reshape(B, D)


# speedup vs baseline: 16.8306x; 1.0224x over previous
import jax
import jax.numpy as jnp
from jax.experimental import pallas as pl
from jax.experimental.pallas import tpu as pltpu

L, D, DI, N, K, DTR = 4, 256, 512, 16, 4, 16
B, T = 4, 1024
BG = 2
TT = BG * T
TC = 64
NCH = T // TC


def _layer_norm(x, w, b):
    m = jnp.mean(x, axis=-1, keepdims=True)
    xc = x - m
    v = jnp.mean(xc * xc, axis=-1, keepdims=True)
    return xc * jax.lax.rsqrt(v + 1e-5) * w + b


def _softplus(z):
    return jnp.maximum(z, 0.0) + jnp.log1p(jnp.exp(-jnp.abs(z)))


def _body(x_ref, lnw_ref, lnb_ref, inw_ref, cw_ref, cb_ref, xw_ref, dtw_ref,
          dtb_ref, alog_ref, dskip_ref, outw_ref, opw_ref, opb_ref, flnw_ref,
          flnb_ref, erep_ref, o_ref,
          u_s, res_s, dt_s, bc_s, upad_s, da_s, dbu_s, hs_s, y_s):
    x = x_ref[...].reshape(TT, D)
    upad_s[:, 0:8, :] = jnp.zeros((BG, 8, DI), jnp.float32)
    erep = erep_ref[...]

    for l in range(L):
        xn = _layer_norm(x, lnw_ref[l], lnb_ref[l])
        ur = jnp.dot(xn, inw_ref[l], preferred_element_type=jnp.float32)
        upad_s[:, 8:, :] = ur[:, :DI].reshape(BG, T, DI)
        res_s[...] = ur[:, DI:]

        cw = cw_ref[l]
        z3 = (cw[3] * upad_s[:, 8:8 + T, :]
              + cw[2] * upad_s[:, 7:7 + T, :]
              + cw[1] * upad_s[:, 6:6 + T, :]
              + cw[0] * upad_s[:, 5:5 + T, :]) + cb_ref[l]
        z = z3.reshape(TT, DI)
        u = z * jax.nn.sigmoid(z)
        u_s[:, :, 0, :] = u.reshape(BG, T, DI)

        del z3, z
        proj = jnp.dot(u, xw_ref[l], preferred_element_type=jnp.float32)
        bc_s[...] = proj.reshape(BG, T, DTR + 2 * N)
        zdt = jnp.dot(proj[:, :DTR], dtw_ref[l],
                      preferred_element_type=jnp.float32) + dtb_ref[l]
        dt_s[:, :, 0, :] = _softplus(zdt).reshape(BG, T, DI)
        At = -jnp.exp(alog_ref[l])

        h = jnp.zeros((BG * N, DI), jnp.float32)
        for c in range(NCH):
            sl = slice(c * TC, (c + 1) * TC)
            dtc = dt_s[:, sl, :, :]
            duc = dtc * u_s[:, sl, :, :]
            bcc = bc_s[:, sl, :]
            da4 = jnp.exp(dtc * At[None, None, :, :])
            da_s[:, 0:N, :] = da4[0]
            da_s[:, N:2 * N, :] = da4[1]
            b4 = jnp.einsum('btn,nmd->btmd', bcc[:, :, DTR:DTR + N], erep,
                            preferred_element_type=jnp.float32)
            dbu4 = duc * b4
            dbu_s[:, 0:N, :] = dbu4[0]
            dbu_s[:, N:2 * N, :] = dbu4[1]

            def scan_step(i, hh):
                for k in range(4):
                    t = i * 4 + k
                    hh = da_s[t] * hh + dbu_s[t]
                    hs_s[t] = hh
                return hh

            h = jax.lax.fori_loop(0, TC // 4, scan_step, h)

            c4 = jnp.einsum('btn,nmd->btmd', bcc[:, :, DTR + N:], erep,
                            preferred_element_type=jnp.float32)
            hsv = hs_s[...]
            y_s[0, sl, :] = jnp.sum(hsv[:, 0:N, :] * c4[0], axis=1)
            y_s[1, sl, :] = jnp.sum(hsv[:, N:2 * N, :] * c4[1], axis=1)

        y = (y_s[...].reshape(TT, DI)
             + u_s[:, :, 0, :].reshape(TT, DI) * dskip_ref[l])
        resv = res_s[...]
        y = y * (resv * jax.nn.sigmoid(resv))
        x = x + jnp.dot(y, outw_ref[l], preferred_element_type=jnp.float32)

    z = jnp.dot(x, opw_ref[...], preferred_element_type=jnp.float32) + opb_ref[...]
    zn = _layer_norm(z, flnw_ref[...], flnb_ref[...])
    o_ref[...] = jnp.mean(zn.reshape(BG, T, D), axis=1, keepdims=True)


def kernel(frame_embeddings, ln_w, ln_b, in_w, conv_w, conv_b, xproj_w,
           dt_w, dt_b, A_log, D_skip, out_w, oproj_w, oproj_b, fln_w, fln_b):
    cwT = conv_w.transpose(0, 2, 1)
    alogT = A_log.transpose(0, 2, 1)
    erep = jnp.broadcast_to(jnp.eye(N, dtype=jnp.float32)[:, :, None],
                            (N, N, DI))

    whole = lambda arr: pl.BlockSpec(arr.shape, lambda b: (0,) * arr.ndim)
    out = pl.pallas_call(
        _body,
        grid=(B // BG,),
        in_specs=[
            pl.BlockSpec((BG, T, D), lambda b: (b, 0, 0)),
            whole(ln_w), whole(ln_b), whole(in_w), whole(cwT), whole(conv_b),
            whole(xproj_w), whole(dt_w), whole(dt_b), whole(alogT),
            whole(D_skip), whole(out_w), whole(oproj_w), whole(oproj_b),
            whole(fln_w), whole(fln_b), whole(erep),
        ],
        out_specs=pl.BlockSpec((BG, 1, D), lambda b: (b, 0, 0)),
        out_shape=jax.ShapeDtypeStruct((B, 1, D), jnp.float32),
        scratch_shapes=[
            pltpu.VMEM((BG, T, 1, DI), jnp.float32),
            pltpu.VMEM((TT, DI), jnp.float32),
            pltpu.VMEM((BG, T, 1, DI), jnp.float32),
            pltpu.VMEM((BG, T, DTR + 2 * N), jnp.float32),
            pltpu.VMEM((BG, T + 8, DI), jnp.float32),
            pltpu.VMEM((TC, BG * N, DI), jnp.float32),
            pltpu.VMEM((TC, BG * N, DI), jnp.float32),
            pltpu.VMEM((TC, BG * N, DI), jnp.float32),
            pltpu.VMEM((BG, T, DI), jnp.float32),
        ],
        compiler_params=pltpu.CompilerParams(
            dimension_semantics=("arbitrary",),
            vmem_limit_bytes=56 * 1024 * 1024,
        ),
        name="mamba_encoder_fused",
    )(frame_embeddings, ln_w, ln_b, in_w, cwT, conv_b, xproj_w, dt_w, dt_b,
      alogT, D_skip, out_w, oproj_w, oproj_b, fln_w, fln_b, erep)
    return out.reshape(B, D)
